# Initial kernel scaffold; baseline (speedup 1.0000x reference)
import jax
import jax.numpy as jnp
from jax.experimental import pallas as pl


def kernel(x, router_w, expert_w, expert_b):
    raise NotImplementedError("write your pallas kernel here")


# speedup vs baseline: 1.6365x; 1.6365x over previous
import functools

import jax
import jax.numpy as jnp
from jax.experimental import pallas as pl
from jax.experimental.pallas import tpu as pltpu

_B, _D, _P, _E, _C = 2048, 2048, 16, 8, 128
_K = _D // _P
_C2 = 2 * _C
_EC2 = _E * _C2
_BB = 256
_NBLK = _B // _BB


def _moe_body(xp_ref, rwt_ref, wall_ref, ball_ref, s_ref,
              out_ref, sel0_ref, stats_ref, loss_ref):
    i = pl.program_id(0)

    xp = xp_ref[...]
    xsum = xp.reshape(_BB, _P, _K).sum(axis=1)
    sel = jnp.dot(xsum, rwt_ref[...],
                  preferred_element_type=jnp.float32)
    gate = jnp.max(sel, axis=1, keepdims=True)
    eiota = jax.lax.broadcasted_iota(jnp.int32, (_BB, _E), 1)
    idx = jnp.min(jnp.where(sel == gate, eiota, _E), axis=1,
                  keepdims=True)
    onehot = (eiota == idx).astype(jnp.float32)
    sel0 = jnp.where(gate != 0.0, onehot, 0.0)
    sel0_ref[...] = sel0

    xb = xp.astype(jnp.bfloat16)
    z = jnp.dot(xb, wall_ref[...],
                preferred_element_type=jnp.float32)
    h = z + ball_ref[...]
    h3 = h * h * h
    hp = h3.reshape(_BB, _P, _EC2).sum(axis=1)
    chunks = jnp.dot(hp, s_ref[...],
                     preferred_element_type=jnp.float32)
    l0 = jnp.sum(chunks[:, :_E] * onehot, axis=1, keepdims=True)
    l1 = jnp.sum(chunks[:, _E:] * onehot, axis=1, keepdims=True)
    a0 = gate * l0
    a1 = gate * l1
    m = jnp.maximum(a0, a1)
    e0 = jnp.exp(a0 - m)
    e1 = jnp.exp(a1 - m)
    denom = e0 + e1
    out_ref[...] = jnp.concatenate([e0 / denom, e1 / denom], axis=1)

    part = jnp.concatenate([jnp.sum(sel, axis=0, keepdims=True),
                            jnp.sum(onehot, axis=0, keepdims=True)],
                           axis=1)
    @pl.when(i == 0)
    def _():
        stats_ref[...] = jnp.zeros_like(stats_ref)
    stats_ref[...] += part

    @pl.when(i == _NBLK - 1)
    def _():
        st = stats_ref[...]
        prod = st[:, :_E] * st[:, _E:]
        loss_ref[...] = (jnp.sum(prod, axis=1, keepdims=True)
                         * (float(_E) / float(_B * _B)))


@functools.partial(jax.jit, static_argnames=())
def _moe_call(xp, rwt, wall, ball, smat):
    out, sel0, stats, loss = pl.pallas_call(
        _moe_body,
        grid=(_NBLK,),
        in_specs=[
            pl.BlockSpec((_BB * _P, _K), lambda i: (i, 0)),
            pl.BlockSpec((_K, _E), lambda i: (0, 0)),
            pl.BlockSpec((_K, _EC2), lambda i: (0, 0)),
            pl.BlockSpec((1, _EC2), lambda i: (0, 0)),
            pl.BlockSpec((_EC2, 2 * _E), lambda i: (0, 0)),
        ],
        out_specs=[
            pl.BlockSpec((_BB, 2), lambda i: (i, 0)),
            pl.BlockSpec((_BB, _E), lambda i: (i, 0)),
            pl.BlockSpec((1, 2 * _E), lambda i: (0, 0)),
            pl.BlockSpec((1, 1), lambda i: (0, 0)),
        ],
        out_shape=[
            jax.ShapeDtypeStruct((_B, 2), jnp.float32),
            jax.ShapeDtypeStruct((_B, _E), jnp.float32),
            jax.ShapeDtypeStruct((1, 2 * _E), jnp.float32),
            jax.ShapeDtypeStruct((1, 1), jnp.float32),
        ],
        compiler_params=pltpu.CompilerParams(
            dimension_semantics=("arbitrary",),
        ),
    )(xp, rwt, wall, ball, smat)
    return out, sel0, loss


def kernel(x, router_w, expert_w, expert_b):
    xp = x.reshape(_B * _P, _K)
    rwt = router_w.T
    wall = (expert_w.reshape(_E, 2, _C, _K)
            .transpose(3, 1, 0, 2).reshape(_K, _EC2)
            .astype(jnp.bfloat16))
    ball = (expert_b.reshape(_E, 2, _C)
            .transpose(1, 0, 2).reshape(1, _EC2))
    smat = (jnp.arange(_EC2)[:, None] // _C == jnp.arange(2 * _E)[None, :]
            ).astype(jnp.float32)
    out, sel0, loss = _moe_call(xp, rwt, wall, ball, smat)
    return out, sel0, loss[0, 0]
